# baseline (speedup 1.0000x reference)
import functools

import jax
import jax.numpy as jnp
import numpy as np
from jax import lax
from jax.experimental import pallas as pl
from jax.experimental.pallas import tpu as pltpu
from jax.experimental.pallas import tpu_sc as plsc

_HIDDEN = 128
_NGAUSS = 50
_NFILT = 128
_NNODES = 10000
_NEDGES = 320000
_CUTOFF = 10.0

_NC = 2
_NS = 16
_NW = _NC * _NS
_CH = 64
_EPW = 10240
_NHALF = 2
_EPH = _EPW // _NHALF
_NCHUNK_H = _EPH // _CH
_NPAIR = _NCHUNK_H // 2
_NE_PAD = _NW * _EPW
_EBLK = 2560
_NEBLK_REAL = _NEDGES // _EBLK
_NEBLK_PAD = _NE_PAD // _EBLK
_ACC_ROWS = 10112
_RPS = 632
_DUMMY = _NNODES


def _edge_mlp(edge_attr, ew2d, w1, b1, w2, b2):
  last = _NEBLK_REAL - 1

  def body(ea, ew, w1r, b1r, w2r, b2r, o):
    hmid = jnp.tanh(
        jnp.dot(ea[...].astype(jnp.bfloat16), w1r[...].astype(jnp.bfloat16),
                preferred_element_type=jnp.float32)
        + b1r[...])
    w = jnp.dot(hmid.astype(jnp.bfloat16), w2r[...].astype(jnp.bfloat16),
                preferred_element_type=jnp.float32) + b2r[...]
    d = ew[0, 0, :]
    cut = 0.5 * (jnp.cos(d * (np.pi / _CUTOFF)) + 1.0) * (
        d < _CUTOFF).astype(jnp.float32)
    o[...] = w * cut[:, None]

  return pl.pallas_call(
      body,
      grid=(_NEBLK_PAD,),
      in_specs=[
          pl.BlockSpec((_EBLK, _NGAUSS), lambda i: (jnp.minimum(i, last), 0)),
          pl.BlockSpec((1, 1, _EBLK), lambda i: (jnp.minimum(i, last), 0, 0)),
          pl.BlockSpec((_NGAUSS, _NFILT), lambda i: (0, 0)),
          pl.BlockSpec((1, _NFILT), lambda i: (0, 0)),
          pl.BlockSpec((_NFILT, _NFILT), lambda i: (0, 0)),
          pl.BlockSpec((1, _NFILT), lambda i: (0, 0)),
      ],
      out_specs=pl.BlockSpec((_EBLK, _NFILT), lambda i: (i, 0)),
      out_shape=jax.ShapeDtypeStruct((_NE_PAD, _NFILT), jnp.float32),
  )(edge_attr, ew2d, w1, b1, w2, b2)


def _node_matmul(x, w):
  def body(xr, wr, o):
    o[...] = jnp.dot(xr[...], wr[...], preferred_element_type=jnp.float32)

  return pl.pallas_call(
      body,
      out_shape=jax.ShapeDtypeStruct((x.shape[0], w.shape[1]), jnp.float32),
  )(x, w)


def _sc_gather_scatter(h_j, w_e, src, dst4):
  mesh = plsc.VectorSubcoreMesh(core_axis_name="c", subcore_axis_name="s")

  @functools.partial(
      pl.kernel,
      mesh=mesh,
      out_type=jax.ShapeDtypeStruct((_NC, _ACC_ROWS, _NFILT), jnp.float32),
      scratch_types=[
          pltpu.VMEM((_EPH,), jnp.int32),
          pltpu.VMEM((_NCHUNK_H, _CH), jnp.int32),
          pltpu.VMEM((_CH, _NFILT), jnp.float32),
          pltpu.VMEM((_CH, _NFILT), jnp.float32),
          pltpu.VMEM((_CH, _NFILT), jnp.float32),
          pltpu.VMEM((_CH, _NFILT), jnp.float32),
          pltpu.VMEM_SHARED((_ACC_ROWS, _NFILT), jnp.float32),
          pltpu.SemaphoreType.DMA,
          pltpu.SemaphoreType.DMA,
          pltpu.SemaphoreType.DMA,
          pltpu.SemaphoreType.DMA,
          pltpu.SemaphoreType.DMA,
          pltpu.SemaphoreType.DMA,
      ],
  )
  def sck(hj_hbm, w_hbm, src_hbm, dst_hbm, out_hbm,
          src_v, dst_v, rows0, rows1, wv0, wv1, acc,
          gsem0, gsem1, wsem0, wsem1, ssem0, ssem1):
    c = lax.axis_index("c")
    s = lax.axis_index("s")
    rows = (rows0, rows1)
    wv = (wv0, wv1)
    gsem = (gsem0, gsem1)
    wsem = (wsem0, wsem1)
    ssem = (ssem0, ssem1)

    def zrow(r, carry):
      for k in range(_NFILT // 16):
        rows0[r, pl.ds(k * 16, 16)] = jnp.zeros((16,), jnp.float32)
      return carry

    lax.fori_loop(0, _CH, zrow, 0)
    nfull = _RPS // _CH
    for t in range(nfull):
      pltpu.sync_copy(rows0, acc.at[pl.ds(s * _RPS + t * _CH, _CH)])
    rem = _RPS - nfull * _CH
    if rem:
      pltpu.sync_copy(rows0.at[pl.ds(0, rem)],
                      acc.at[pl.ds(s * _RPS + nfull * _CH, rem)])
    plsc.subcore_barrier()

    wid = c * _NS + s

    def g_issue(k, b):
      return pltpu.async_copy(
          hj_hbm.at[src_v.at[pl.ds(k * _CH, _CH)]], rows[b], gsem[b])

    def w_issue(hbase, k, b):
      return pltpu.async_copy(
          w_hbm.at[pl.ds(hbase + k * _CH, _CH)], wv[b], wsem[b])

    def s_issue(k, b):
      return pltpu.async_copy(wv[b], acc.at[dst_v.at[k]], ssem[b], add=True)

    def s_wait(k, b):
      pltpu.make_async_copy(wv[b], acc.at[dst_v.at[k]], ssem[b]).wait()

    for h in range(_NHALF):
      hbase = wid * _EPW + h * _EPH
      pltpu.sync_copy(src_hbm.at[pl.ds(hbase, _EPH)], src_v)
      pltpu.sync_copy(dst_hbm.at[wid].at[h], dst_v)
      g_issue(0, 0)
      g_issue(1, 1)
      w_issue(hbase, 0, 0)

      def pair(kk, carry):
        for b in range(2):
          b2 = 1 - b
          k = kk * 2 + b
          pltpu.make_async_copy(
              hj_hbm.at[src_v.at[pl.ds(k * _CH, _CH)]], rows[b],
              gsem[b]).wait()
          pltpu.make_async_copy(
              w_hbm.at[pl.ds(hbase + k * _CH, _CH)], wv[b], wsem[b]).wait()

          def row(r, rc):
            for kk8 in range(_NFILT // 16):
              sl = pl.ds(kk8 * 16, 16)
              wv[b][r, sl] = rows[b][r, sl] * wv[b][r, sl]
            return rc

          lax.fori_loop(0, _CH, row, 0)

          @pl.when(kk < _NPAIR - 1)
          def _():
            g_issue(k + 2, b)

          s_issue(k, b)
          if b == 0:
            @pl.when(kk > 0)
            def _():
              s_wait(k - 1, b2)

            w_issue(hbase, k + 1, b2)
          else:
            s_wait(k - 1, b2)

            @pl.when(kk < _NPAIR - 1)
            def _():
              w_issue(hbase, k + 1, b2)
        return carry

      lax.fori_loop(0, _NPAIR, pair, 0)
      s_wait(2 * _NPAIR - 1, 1)

    plsc.subcore_barrier()
    pltpu.sync_copy(acc.at[pl.ds(s * _RPS, _RPS)],
                    out_hbm.at[c].at[pl.ds(s * _RPS, _RPS)])

  return sck(h_j, w_e, src, dst4)


def _final(partial, lout_w, lout_b):
  nblk = 25
  rows = _NNODES // nblk

  def body(p, lw, lb, o):
    pp = p[...]
    o[...] = jnp.dot(pp[0] + pp[1], lw[...],
                     preferred_element_type=jnp.float32) + lb[...]

  return pl.pallas_call(
      body,
      grid=(nblk,),
      in_specs=[
          pl.BlockSpec((_NC, rows, _NFILT), lambda i: (0, i, 0)),
          pl.BlockSpec((_NFILT, _HIDDEN), lambda i: (0, 0)),
          pl.BlockSpec((1, _HIDDEN), lambda i: (0, 0)),
      ],
      out_specs=pl.BlockSpec((rows, _HIDDEN), lambda i: (i, 0)),
      out_shape=jax.ShapeDtypeStruct((_NNODES, _HIDDEN), jnp.float32),
  )(partial, lout_w, lout_b)


def kernel(h, edge_index, edge_weight, edge_attr,
           mlp_W1, mlp_b1, mlp_W2, mlp_b2, lin_W, lout_W, lout_b):
  src = edge_index[0].astype(jnp.int32)
  dst = edge_index[1].astype(jnp.int32)
  npad = _NE_PAD - _NEDGES
  src_p = jnp.pad(src, (0, npad))
  dst_p = jnp.pad(dst, (0, npad), constant_values=_DUMMY)
  dst4 = dst_p.reshape(_NW, _NHALF, _NCHUNK_H, _CH)
  ew3d = edge_weight.reshape(_NEBLK_REAL, 1, _EBLK)

  w_e = _edge_mlp(edge_attr, ew3d, mlp_W1, mlp_b1.reshape(1, -1),
                  mlp_W2, mlp_b2.reshape(1, -1))
  h_j = _node_matmul(h, lin_W)
  partial = _sc_gather_scatter(h_j, w_e, src_p, dst4)
  return _final(partial, lout_W, lout_b.reshape(1, -1))


# speedup vs baseline: 1.0222x; 1.0222x over previous
import functools

import jax
import jax.numpy as jnp
import numpy as np
from jax import lax
from jax.experimental import pallas as pl
from jax.experimental.pallas import tpu as pltpu
from jax.experimental.pallas import tpu_sc as plsc

_HIDDEN = 128
_NGAUSS = 50
_NFILT = 128
_NNODES = 10000
_NEDGES = 320000
_CUTOFF = 10.0

_NC = 2
_NS = 16
_NW = _NC * _NS
_CH = 64
_NPH = 2
_EPW = 5120
_NHALF = 2
_EPH = _EPW // _NHALF
_NCHUNK_H = _EPH // _CH
_NPAIR = _NCHUNK_H // 2
_NE_PH = _NW * _EPW
_NE_PAD = _NPH * _NE_PH
_EBLK = 2560
_NEBLK_REAL = _NEDGES // _EBLK
_NEBLK_PAD = _NE_PAD // _EBLK
_ACC_ROWS = 10112
_RPS = 632
_DUMMY = _NNODES


def _edge_mlp(edge_attr, ew2d, w1, b1, w2, b2, nblk_real, nblk_pad):
  last = nblk_real - 1

  def body(ea, ew, w1r, b1r, w2r, b2r, o):
    hmid = jnp.tanh(
        jnp.dot(ea[...].astype(jnp.bfloat16), w1r[...].astype(jnp.bfloat16),
                preferred_element_type=jnp.float32)
        + b1r[...])
    w = jnp.dot(hmid.astype(jnp.bfloat16), w2r[...].astype(jnp.bfloat16),
                preferred_element_type=jnp.float32) + b2r[...]
    d = ew[0, 0, :]
    cut = 0.5 * (jnp.cos(d * (np.pi / _CUTOFF)) + 1.0) * (
        d < _CUTOFF).astype(jnp.float32)
    o[...] = w * cut[:, None]

  return pl.pallas_call(
      body,
      grid=(nblk_pad,),
      in_specs=[
          pl.BlockSpec((_EBLK, _NGAUSS), lambda i: (jnp.minimum(i, last), 0)),
          pl.BlockSpec((1, 1, _EBLK), lambda i: (jnp.minimum(i, last), 0, 0)),
          pl.BlockSpec((_NGAUSS, _NFILT), lambda i: (0, 0)),
          pl.BlockSpec((1, _NFILT), lambda i: (0, 0)),
          pl.BlockSpec((_NFILT, _NFILT), lambda i: (0, 0)),
          pl.BlockSpec((1, _NFILT), lambda i: (0, 0)),
      ],
      out_specs=pl.BlockSpec((_EBLK, _NFILT), lambda i: (i, 0)),
      out_shape=jax.ShapeDtypeStruct((nblk_pad * _EBLK, _NFILT), jnp.float32),
  )(edge_attr, ew2d, w1, b1, w2, b2)


def _node_matmul(x, w):
  def body(xr, wr, o):
    o[...] = jnp.dot(xr[...], wr[...], preferred_element_type=jnp.float32)

  return pl.pallas_call(
      body,
      out_shape=jax.ShapeDtypeStruct((x.shape[0], w.shape[1]), jnp.float32),
  )(x, w)


def _sc_gather_scatter(h_j, w_e, src, dst4):
  mesh = plsc.VectorSubcoreMesh(core_axis_name="c", subcore_axis_name="s")

  @functools.partial(
      pl.kernel,
      mesh=mesh,
      out_type=jax.ShapeDtypeStruct((_NC, _ACC_ROWS, _NFILT), jnp.float32),
      scratch_types=[
          pltpu.VMEM((_EPH,), jnp.int32),
          pltpu.VMEM((_NCHUNK_H, _CH), jnp.int32),
          pltpu.VMEM((_CH, _NFILT), jnp.float32),
          pltpu.VMEM((_CH, _NFILT), jnp.float32),
          pltpu.VMEM((_CH, _NFILT), jnp.float32),
          pltpu.VMEM((_CH, _NFILT), jnp.float32),
          pltpu.VMEM_SHARED((_ACC_ROWS, _NFILT), jnp.float32),
          pltpu.SemaphoreType.DMA,
          pltpu.SemaphoreType.DMA,
          pltpu.SemaphoreType.DMA,
          pltpu.SemaphoreType.DMA,
          pltpu.SemaphoreType.DMA,
          pltpu.SemaphoreType.DMA,
      ],
  )
  def sck(hj_hbm, w_hbm, src_hbm, dst_hbm, out_hbm,
          src_v, dst_v, rows0, rows1, wv0, wv1, acc,
          gsem0, gsem1, wsem0, wsem1, ssem0, ssem1):
    c = lax.axis_index("c")
    s = lax.axis_index("s")
    rows = (rows0, rows1)
    wv = (wv0, wv1)
    gsem = (gsem0, gsem1)
    wsem = (wsem0, wsem1)
    ssem = (ssem0, ssem1)

    def zrow(r, carry):
      for k in range(_NFILT // 16):
        rows0[r, pl.ds(k * 16, 16)] = jnp.zeros((16,), jnp.float32)
      return carry

    lax.fori_loop(0, _CH, zrow, 0)
    nfull = _RPS // _CH
    for t in range(nfull):
      pltpu.sync_copy(rows0, acc.at[pl.ds(s * _RPS + t * _CH, _CH)])
    rem = _RPS - nfull * _CH
    if rem:
      pltpu.sync_copy(rows0.at[pl.ds(0, rem)],
                      acc.at[pl.ds(s * _RPS + nfull * _CH, rem)])
    plsc.subcore_barrier()

    wid = c * _NS + s

    def g_issue(k, b):
      return pltpu.async_copy(
          hj_hbm.at[src_v.at[pl.ds(k * _CH, _CH)]], rows[b], gsem[b])

    def w_issue(hbase, k, b):
      return pltpu.async_copy(
          w_hbm.at[pl.ds(hbase + k * _CH, _CH)], wv[b], wsem[b])

    def s_issue(k, b):
      return pltpu.async_copy(wv[b], acc.at[dst_v.at[k]], ssem[b], add=True)

    def s_wait(k, b):
      pltpu.make_async_copy(wv[b], acc.at[dst_v.at[k]], ssem[b]).wait()

    for h in range(_NHALF):
      hbase = wid * _EPW + h * _EPH
      pltpu.sync_copy(src_hbm.at[pl.ds(hbase, _EPH)], src_v)
      pltpu.sync_copy(dst_hbm.at[wid].at[h], dst_v)
      g_issue(0, 0)
      g_issue(1, 1)
      w_issue(hbase, 0, 0)

      def pair(kk, carry):
        for b in range(2):
          b2 = 1 - b
          k = kk * 2 + b
          pltpu.make_async_copy(
              hj_hbm.at[src_v.at[pl.ds(k * _CH, _CH)]], rows[b],
              gsem[b]).wait()
          pltpu.make_async_copy(
              w_hbm.at[pl.ds(hbase + k * _CH, _CH)], wv[b], wsem[b]).wait()

          def row(r, rc):
            for kk8 in range(_NFILT // 16):
              sl = pl.ds(kk8 * 16, 16)
              wv[b][r, sl] = rows[b][r, sl] * wv[b][r, sl]
            return rc

          lax.fori_loop(0, _CH, row, 0)

          @pl.when(kk < _NPAIR - 1)
          def _():
            g_issue(k + 2, b)

          s_issue(k, b)
          if b == 0:
            @pl.when(kk > 0)
            def _():
              s_wait(k - 1, b2)

            w_issue(hbase, k + 1, b2)
          else:
            s_wait(k - 1, b2)

            @pl.when(kk < _NPAIR - 1)
            def _():
              w_issue(hbase, k + 1, b2)
        return carry

      lax.fori_loop(0, _NPAIR, pair, 0)
      s_wait(2 * _NPAIR - 1, 1)

    plsc.subcore_barrier()
    pltpu.sync_copy(acc.at[pl.ds(s * _RPS, _RPS)],
                    out_hbm.at[c].at[pl.ds(s * _RPS, _RPS)])

  return sck(h_j, w_e, src, dst4)


def _final(part_a, part_b, lout_w, lout_b):
  nblk = 25
  rows = _NNODES // nblk

  def body(p, q, lw, lb, o):
    pp = p[...]
    qq = q[...]
    o[...] = jnp.dot(pp[0] + pp[1] + qq[0] + qq[1], lw[...],
                     preferred_element_type=jnp.float32) + lb[...]

  return pl.pallas_call(
      body,
      grid=(nblk,),
      in_specs=[
          pl.BlockSpec((_NC, rows, _NFILT), lambda i: (0, i, 0)),
          pl.BlockSpec((_NC, rows, _NFILT), lambda i: (0, i, 0)),
          pl.BlockSpec((_NFILT, _HIDDEN), lambda i: (0, 0)),
          pl.BlockSpec((1, _HIDDEN), lambda i: (0, 0)),
      ],
      out_specs=pl.BlockSpec((rows, _HIDDEN), lambda i: (i, 0)),
      out_shape=jax.ShapeDtypeStruct((_NNODES, _HIDDEN), jnp.float32),
  )(part_a, part_b, lout_w, lout_b)


def kernel(h, edge_index, edge_weight, edge_attr,
           mlp_W1, mlp_b1, mlp_W2, mlp_b2, lin_W, lout_W, lout_b):
  src = edge_index[0].astype(jnp.int32)
  dst = edge_index[1].astype(jnp.int32)
  npad = _NE_PAD - _NEDGES
  src_p = jnp.pad(src, (0, npad))
  dst_p = jnp.pad(dst, (0, npad), constant_values=_DUMMY)
  dst5 = dst_p.reshape(_NPH, _NW, _NHALF, _NCHUNK_H, _CH)

  h_j = _node_matmul(h, lin_W)
  b1r = mlp_b1.reshape(1, -1)
  b2r = mlp_b2.reshape(1, -1)
  nblk_pad = _NE_PH // _EBLK
  parts = []
  for p in range(_NPH):
    lo = p * _NE_PH
    hi = min((p + 1) * _NE_PH, _NEDGES)
    nblk_real = (hi - lo) // _EBLK
    ea_p = edge_attr[lo:hi]
    ew_p = edge_weight[lo:hi].reshape(nblk_real, 1, _EBLK)
    w_p = _edge_mlp(ea_p, ew_p, mlp_W1, b1r, mlp_W2, b2r,
                    nblk_real, nblk_pad)
    parts.append(_sc_gather_scatter(
        h_j, w_p, src_p[lo:lo + _NE_PH], dst5[p]))
  return _final(parts[0], parts[1], lout_W, lout_b.reshape(1, -1))
